# baseline (speedup 1.0000x reference)
import functools

import jax
import jax.numpy as jnp
from jax import lax
from jax.experimental import pallas as pl
from jax.experimental.pallas import tpu as pltpu
from jax.experimental.pallas import tpu_sc as plsc

NC = 2
NS = 16
NW = NC * NS


_EB = 6400


def _k1_body(ei_ref, x_ref, wa_ref, ba_ref, ea_ref, wb_ref, bb_ref,
             src_ref, dst_ref, h_ref, e_ref, *, hprog):
  i = pl.program_id(0)

  @pl.when(i == 0)
  def _split():
    src_ref[...] = ei_ref[0, :]
    dst_ref[...] = ei_ref[1, :]

  e_ref[...] = (
      jnp.dot(ea_ref[...], wb_ref[...], preferred_element_type=jnp.float32)
      + bb_ref[...]
  )

  @pl.when(i < hprog)
  def _():
    h_ref[...] = (
        jnp.dot(x_ref[...], wa_ref[...], preferred_element_type=jnp.float32)
        + ba_ref[...]
    )


def _k1(edge_index, x, wa, ba, ea, wb, bb):
  n, d = x.shape
  e_cnt, k = ea.shape
  grid = (e_cnt // _EB,)
  hb = 2000
  hprog = n // hb
  assert n % hb == 0 and hprog <= grid[0]

  def hmap(i):
    return (jnp.minimum(i, hprog - 1), 0)

  return pl.pallas_call(
      functools.partial(_k1_body, hprog=hprog),
      out_shape=(
          jax.ShapeDtypeStruct((e_cnt,), jnp.int32),
          jax.ShapeDtypeStruct((e_cnt,), jnp.int32),
          jax.ShapeDtypeStruct((n, d), jnp.float32),
          jax.ShapeDtypeStruct((e_cnt, d), jnp.float32),
      ),
      grid=grid,
      in_specs=[
          pl.BlockSpec((2, e_cnt), lambda i: (0, 0)),
          pl.BlockSpec((hb, d), hmap),
          pl.BlockSpec((d, d), lambda i: (0, 0)),
          pl.BlockSpec((1, d), lambda i: (0, 0)),
          pl.BlockSpec((_EB, k), lambda i: (i, 0)),
          pl.BlockSpec((k, d), lambda i: (0, 0)),
          pl.BlockSpec((1, d), lambda i: (0, 0)),
      ],
      out_specs=(
          pl.BlockSpec((e_cnt,), lambda i: (0,)),
          pl.BlockSpec((e_cnt,), lambda i: (0,)),
          pl.BlockSpec((hb, d), hmap),
          pl.BlockSpec((_EB, d), lambda i: (i, 0)),
      ),
  )(edge_index, x, wa, ba, ea, wb, bb)


def _bn(z, g_ref, b_ref):
  mu = jnp.mean(z, axis=0, keepdims=True)
  var = jnp.mean((z - mu) * (z - mu), axis=0, keepdims=True)
  return g_ref[...] * (z - mu) / jnp.sqrt(var + 1e-5) + b_ref[...]


def _mlp_body(h_ref, agg_ref, eps_ref, w1_ref, b1_ref, gm_ref, bm_ref,
              w2_ref, b2_ref, g_ref, b_ref, o_ref, *, final_relu):
  agg = agg_ref[0] + agg_ref[1]
  z = (1.0 + eps_ref[0, 0]) * h_ref[...] + agg
  z1 = jnp.dot(z, w1_ref[...], preferred_element_type=jnp.float32) + b1_ref[...]
  z1 = jnp.maximum(_bn(z1, gm_ref, bm_ref), 0.0)
  z2 = jnp.dot(z1, w2_ref[...], preferred_element_type=jnp.float32) + b2_ref[...]
  z2 = _bn(z2, g_ref, b_ref)
  if final_relu:
    z2 = jnp.maximum(z2, 0.0)
  o_ref[...] = z2


def _node_mlp(h, agg2, eps, w1, b1, gm, bm, w2, b2, g, b, final_relu):
  n, d = h.shape
  return pl.pallas_call(
      functools.partial(_mlp_body, final_relu=final_relu),
      out_shape=jax.ShapeDtypeStruct((n, d), jnp.float32),
  )(h, agg2, eps, w1, b1, gm, bm, w2, b2, g, b)


def _mlp_eproj_body(h_ref, agg_ref, eps_ref, w1_ref, b1_ref, gm_ref, bm_ref,
                    w2_ref, b2_ref, g_ref, b_ref, ea_ref, wb_ref, bb_ref,
                    hn_ref, e_ref, *, final_relu, nprog):
  i = pl.program_id(0)
  e_ref[...] = (
      jnp.dot(ea_ref[...], wb_ref[...], preferred_element_type=jnp.float32)
      + bb_ref[...]
  )

  @pl.when(i == nprog - 1)
  def _():
    _mlp_body(h_ref, agg_ref, eps_ref, w1_ref, b1_ref, gm_ref, bm_ref,
              w2_ref, b2_ref, g_ref, b_ref, hn_ref, final_relu=final_relu)


def _node_mlp_eproj(h, agg2, eps, w1, b1, gm, bm, w2, b2, g, b,
                    ea, wb, bb, final_relu):
  n, d = h.shape
  e_cnt, k = ea.shape
  grid = (e_cnt // _EB,)
  full = lambda i: (0, 0)
  return pl.pallas_call(
      functools.partial(_mlp_eproj_body, final_relu=final_relu,
                        nprog=grid[0]),
      out_shape=(
          jax.ShapeDtypeStruct((n, d), jnp.float32),
          jax.ShapeDtypeStruct((e_cnt, d), jnp.float32),
      ),
      grid=grid,
      in_specs=[
          pl.BlockSpec((n, d), full),
          pl.BlockSpec((2, n, d), lambda i: (0, 0, 0)),
          pl.BlockSpec((1, 1), full),
          pl.BlockSpec((d, 2 * d), full),
          pl.BlockSpec((1, 2 * d), full),
          pl.BlockSpec((1, 2 * d), full),
          pl.BlockSpec((1, 2 * d), full),
          pl.BlockSpec((2 * d, d), full),
          pl.BlockSpec((1, d), full),
          pl.BlockSpec((1, d), full),
          pl.BlockSpec((1, d), full),
          pl.BlockSpec((_EB, k), lambda i: (i, 0)),
          pl.BlockSpec((k, d), full),
          pl.BlockSpec((1, d), full),
      ],
      out_specs=(
          pl.BlockSpec((n, d), full),
          pl.BlockSpec((_EB, d), lambda i: (i, 0)),
      ),
  )(h, agg2, eps, w1, b1, gm, bm, w2, b2, g, b, ea, wb, bb)


def _make_edge_pass(n_nodes, n_edges, d, chunk):
  epw = n_edges // NW
  nchunk = epw // chunk
  assert epw * NW == n_edges and nchunk * chunk == epw
  assert chunk % 8 == 0 and chunk <= 128
  assert nchunk >= 8
  zero_tiles = 10
  rows_per_tile = n_nodes // zero_tiles
  assert rows_per_tile * zero_tiles == n_nodes and rows_per_tile % 8 == 0

  mesh = plsc.VectorSubcoreMesh(core_axis_name="c", subcore_axis_name="s")

  @functools.partial(
      pl.kernel,
      out_type=jax.ShapeDtypeStruct((NC, n_nodes, d), jnp.float32),
      mesh=mesh,
      scratch_types=[
          [pltpu.VMEM((chunk,), jnp.int32)] * 4,
          [pltpu.VMEM((chunk,), jnp.int32)] * 4,
          [pltpu.VMEM((chunk, d), jnp.float32)] * 4,
          pltpu.VMEM_SHARED((n_nodes, d), jnp.float32),
          [pltpu.SemaphoreType.DMA] * 4,
          [pltpu.SemaphoreType.DMA] * 4,
          [pltpu.SemaphoreType.DMA] * 4,
          [pltpu.SemaphoreType.DMA] * 4,
      ],
  )
  def edge_pass(h_hbm, e_hbm, src_hbm, dst_hbm, zero_hbm, out_hbm,
                src_v, dst_v, rows_v, agg_sh, isem, gsem, esem, ssem):
    cid = lax.axis_index("c")
    sid = lax.axis_index("s")
    wid = cid * NS + sid

    zrow = sid * rows_per_tile

    @pl.when(sid < zero_tiles)
    def _zero():
      pltpu.sync_copy(zero_hbm, agg_sh.at[pl.ds(zrow, rows_per_tile)])

    plsc.subcore_barrier()

    ebase = wid * epw

    def fire_idx(t, r):
      off = ebase + t * chunk
      pltpu.async_copy(src_hbm.at[pl.ds(off, chunk)], src_v[r], isem[r])
      pltpu.async_copy(dst_hbm.at[pl.ds(off, chunk)], dst_v[r], isem[r])

    def fire_e(t, r):
      pltpu.async_copy(e_hbm.at[pl.ds(ebase + t * chunk, chunk)],
                       rows_v[r], esem[r])

    def fire_ga(r):
      pltpu.make_async_copy(src_hbm.at[pl.ds(0, chunk)], src_v[r],
                            isem[r]).wait()
      pltpu.make_async_copy(dst_hbm.at[pl.ds(0, chunk)], dst_v[r],
                            isem[r]).wait()
      pltpu.make_async_copy(e_hbm.at[pl.ds(0, chunk)], rows_v[r],
                            esem[r]).wait()
      pltpu.async_copy(h_hbm.at[src_v[r]], rows_v[r], gsem[r], add=True)

    def compute(r):
      def row_body(i, c2):
        for u in range(4):
          for v in range(d // 16):
            s = pl.ds(v * 16, 16)
            rows_v[r][i * 4 + u, s] = jnp.maximum(rows_v[r][i * 4 + u, s],
                                                  0.0)
        return c2

      lax.fori_loop(0, chunk // 4, row_body, 0, unroll=False)

    def step(t, k):
      r = k % 4
      if not isinstance(t, int) or t >= 2:
        pltpu.make_async_copy(rows_v[(k + 2) % 4],
                              agg_sh.at[dst_v[(k + 2) % 4]],
                              ssem[(k + 2) % 4]).wait()
      if not isinstance(t, int) or t + 2 < nchunk:
        fire_idx(t + 2, (k + 2) % 4)
        fire_e(t + 2, (k + 2) % 4)
      if not isinstance(t, int) or t + 1 < nchunk:
        fire_ga((k + 1) % 4)
      pltpu.make_async_copy(h_hbm.at[src_v[r]], rows_v[r], gsem[r]).wait()
      compute(r)
      pltpu.async_copy(rows_v[r], agg_sh.at[dst_v[r]], ssem[r], add=True)

    fire_idx(0, 0)
    fire_e(0, 0)
    fire_idx(1, 1)
    fire_e(1, 1)
    fire_ga(0)
    step(0, 0)
    step(1, 1)

    nquad = (nchunk - 4) // 4

    def quad_body(i, carry):
      t = 2 + 4 * i
      step(t, 2)
      step(t + 1, 3)
      step(t + 2, 0)
      step(t + 3, 1)
      return carry

    lax.fori_loop(0, nquad, quad_body, 0, unroll=False)

    for t in range(2 + 4 * nquad, nchunk):
      step(t, t % 4)

    pltpu.make_async_copy(rows_v[(nchunk - 2) % 4],
                          agg_sh.at[dst_v[(nchunk - 2) % 4]],
                          ssem[(nchunk - 2) % 4]).wait()
    pltpu.make_async_copy(rows_v[(nchunk - 1) % 4],
                          agg_sh.at[dst_v[(nchunk - 1) % 4]],
                          ssem[(nchunk - 1) % 4]).wait()

    plsc.subcore_barrier()

    @pl.when(sid < zero_tiles)
    def _dump():
      pltpu.sync_copy(agg_sh.at[pl.ds(zrow, rows_per_tile)],
                      out_hbm.at[cid].at[pl.ds(zrow, rows_per_tile)])

  return edge_pass


def kernel(x, edge_index, edge_attr, batch, params):
  n, d = x.shape
  e_cnt = edge_index.shape[1]
  zero_blk = jnp.zeros((n // 10, d), jnp.float32)

  edge_pass = _make_edge_pass(n, e_cnt, d, chunk=80)

  def mlp_params(l):
    return (
        params['eps%d' % l].reshape(1, 1),
        params['W1_%d' % l], params['b1_%d' % l].reshape(1, -1),
        params['gm%d' % l].reshape(1, -1), params['bm%d' % l].reshape(1, -1),
        params['W2_%d' % l], params['b2_%d' % l].reshape(1, -1),
        params['g%d' % l].reshape(1, -1), params['b%d' % l].reshape(1, -1),
    )

  src, dst, h, e0 = _k1(
      edge_index, x, params['W_atom'], params['b_atom'].reshape(1, -1),
      edge_attr, params['Wb0'], params['bb0'].reshape(1, -1))
  agg2 = edge_pass(h, e0, src, dst, zero_blk)
  h, e1 = _node_mlp_eproj(
      h, agg2, *mlp_params(0),
      edge_attr, params['Wb1'], params['bb1'].reshape(1, -1),
      final_relu=True)
  agg2 = edge_pass(h, e1, src, dst, zero_blk)
  return _node_mlp(h, agg2, *mlp_params(1), final_relu=False)


# speedup vs baseline: 1.0416x; 1.0416x over previous
import functools

import jax
import jax.numpy as jnp
from jax import lax
from jax.experimental import pallas as pl
from jax.experimental.pallas import tpu as pltpu
from jax.experimental.pallas import tpu_sc as plsc

NC = 2
NS = 16
NW = NC * NS


_EB = 6400


def _k1_body(ei_ref, x_ref, wa_ref, ba_ref, ea_ref, wb_ref, bb_ref,
             src_ref, dst_ref, h_ref, e_ref, *, hprog):
  i = pl.program_id(0)

  @pl.when(i == 0)
  def _split():
    src_ref[...] = ei_ref[0, :]
    dst_ref[...] = ei_ref[1, :]

  e_ref[...] = (
      jnp.dot(ea_ref[...], wb_ref[...], preferred_element_type=jnp.float32)
      + bb_ref[...]
  )

  @pl.when(i < hprog)
  def _():
    h_ref[...] = (
        jnp.dot(x_ref[...], wa_ref[...], preferred_element_type=jnp.float32)
        + ba_ref[...]
    )


def _k1(edge_index, x, wa, ba, ea, wb, bb):
  n, d = x.shape
  e_cnt, k = ea.shape
  grid = (e_cnt // _EB,)
  hb = 2000
  hprog = n // hb
  assert n % hb == 0 and hprog <= grid[0]

  def hmap(i):
    return (jnp.minimum(i, hprog - 1), 0)

  return pl.pallas_call(
      functools.partial(_k1_body, hprog=hprog),
      out_shape=(
          jax.ShapeDtypeStruct((e_cnt,), jnp.int32),
          jax.ShapeDtypeStruct((e_cnt,), jnp.int32),
          jax.ShapeDtypeStruct((n, d), jnp.float32),
          jax.ShapeDtypeStruct((e_cnt, d), jnp.float32),
      ),
      grid=grid,
      in_specs=[
          pl.BlockSpec((2, e_cnt), lambda i: (0, 0)),
          pl.BlockSpec((hb, d), hmap),
          pl.BlockSpec((d, d), lambda i: (0, 0)),
          pl.BlockSpec((1, d), lambda i: (0, 0)),
          pl.BlockSpec((_EB, k), lambda i: (i, 0)),
          pl.BlockSpec((k, d), lambda i: (0, 0)),
          pl.BlockSpec((1, d), lambda i: (0, 0)),
      ],
      out_specs=(
          pl.BlockSpec((e_cnt,), lambda i: (0,)),
          pl.BlockSpec((e_cnt,), lambda i: (0,)),
          pl.BlockSpec((hb, d), hmap),
          pl.BlockSpec((_EB, d), lambda i: (i, 0)),
      ),
  )(edge_index, x, wa, ba, ea, wb, bb)


def _bn(z, g_ref, b_ref):
  mu = jnp.mean(z, axis=0, keepdims=True)
  var = jnp.mean((z - mu) * (z - mu), axis=0, keepdims=True)
  return g_ref[...] * (z - mu) / jnp.sqrt(var + 1e-5) + b_ref[...]


def _mlp_body(h_ref, agg_ref, eps_ref, w1_ref, b1_ref, gm_ref, bm_ref,
              w2_ref, b2_ref, g_ref, b_ref, o_ref, *, final_relu):
  agg = agg_ref[0] + agg_ref[1]
  z = (1.0 + eps_ref[0, 0]) * h_ref[...] + agg
  z1 = jnp.dot(z, w1_ref[...], preferred_element_type=jnp.float32) + b1_ref[...]
  z1 = jnp.maximum(_bn(z1, gm_ref, bm_ref), 0.0)
  z2 = jnp.dot(z1, w2_ref[...], preferred_element_type=jnp.float32) + b2_ref[...]
  z2 = _bn(z2, g_ref, b_ref)
  if final_relu:
    z2 = jnp.maximum(z2, 0.0)
  o_ref[...] = z2


def _node_mlp(h, agg2, eps, w1, b1, gm, bm, w2, b2, g, b, final_relu):
  n, d = h.shape
  return pl.pallas_call(
      functools.partial(_mlp_body, final_relu=final_relu),
      out_shape=jax.ShapeDtypeStruct((n, d), jnp.float32),
  )(h, agg2, eps, w1, b1, gm, bm, w2, b2, g, b)


def _eproj_body(ea_ref, wb_ref, bb_ref, e_ref):
  e_ref[...] = (
      jnp.dot(ea_ref[...], wb_ref[...], preferred_element_type=jnp.float32)
      + bb_ref[...]
  )


def _eproj(ea, wb, bb):
  e_cnt, k = ea.shape
  d = wb.shape[1]
  return pl.pallas_call(
      _eproj_body,
      out_shape=jax.ShapeDtypeStruct((e_cnt, d), jnp.float32),
      grid=(e_cnt // _EB,),
      in_specs=[
          pl.BlockSpec((_EB, k), lambda i: (i, 0)),
          pl.BlockSpec((k, d), lambda i: (0, 0)),
          pl.BlockSpec((1, d), lambda i: (0, 0)),
      ],
      out_specs=pl.BlockSpec((_EB, d), lambda i: (i, 0)),
  )(ea, wb, bb)


def _make_edge_pass(n_nodes, n_edges, d, chunk):
  epw = n_edges // NW
  nchunk = epw // chunk
  assert epw * NW == n_edges and nchunk * chunk == epw
  assert chunk % 8 == 0 and chunk <= 128
  assert nchunk >= 8
  zero_tiles = 10
  rows_per_tile = n_nodes // zero_tiles
  assert rows_per_tile * zero_tiles == n_nodes and rows_per_tile % 8 == 0

  mesh = plsc.VectorSubcoreMesh(core_axis_name="c", subcore_axis_name="s")

  @functools.partial(
      pl.kernel,
      out_type=jax.ShapeDtypeStruct((NC, n_nodes, d), jnp.float32),
      mesh=mesh,
      scratch_types=[
          [pltpu.VMEM((chunk,), jnp.int32)] * 4,
          [pltpu.VMEM((chunk,), jnp.int32)] * 4,
          [pltpu.VMEM((chunk, d), jnp.float32)] * 4,
          pltpu.VMEM_SHARED((n_nodes, d), jnp.float32),
          [pltpu.SemaphoreType.DMA] * 4,
          [pltpu.SemaphoreType.DMA] * 4,
          [pltpu.SemaphoreType.DMA] * 4,
          [pltpu.SemaphoreType.DMA] * 4,
      ],
  )
  def edge_pass(h_hbm, e_hbm, src_hbm, dst_hbm, zero_hbm, out_hbm,
                src_v, dst_v, rows_v, agg_sh, isem, gsem, esem, ssem):
    cid = lax.axis_index("c")
    sid = lax.axis_index("s")
    wid = cid * NS + sid

    zrow = sid * rows_per_tile

    @pl.when(sid < zero_tiles)
    def _zero():
      pltpu.sync_copy(zero_hbm, agg_sh.at[pl.ds(zrow, rows_per_tile)])

    plsc.subcore_barrier()

    ebase = wid * epw

    def fire_idx(t, r):
      off = ebase + t * chunk
      pltpu.async_copy(src_hbm.at[pl.ds(off, chunk)], src_v[r], isem[r])
      pltpu.async_copy(dst_hbm.at[pl.ds(off, chunk)], dst_v[r], isem[r])

    def fire_e(t, r):
      pltpu.async_copy(e_hbm.at[pl.ds(ebase + t * chunk, chunk)],
                       rows_v[r], esem[r])

    def fire_ga(r):
      pltpu.make_async_copy(src_hbm.at[pl.ds(0, chunk)], src_v[r],
                            isem[r]).wait()
      pltpu.make_async_copy(dst_hbm.at[pl.ds(0, chunk)], dst_v[r],
                            isem[r]).wait()
      pltpu.make_async_copy(e_hbm.at[pl.ds(0, chunk)], rows_v[r],
                            esem[r]).wait()
      pltpu.async_copy(h_hbm.at[src_v[r]], rows_v[r], gsem[r], add=True)

    def compute(r):
      def row_body(i, c2):
        for u in range(4):
          for v in range(d // 16):
            s = pl.ds(v * 16, 16)
            rows_v[r][i * 4 + u, s] = jnp.maximum(rows_v[r][i * 4 + u, s],
                                                  0.0)
        return c2

      lax.fori_loop(0, chunk // 4, row_body, 0, unroll=False)

    def step(t, k):
      r = k % 4
      if not isinstance(t, int) or t >= 2:
        pltpu.make_async_copy(rows_v[(k + 2) % 4],
                              agg_sh.at[dst_v[(k + 2) % 4]],
                              ssem[(k + 2) % 4]).wait()
      if not isinstance(t, int) or t + 2 < nchunk:
        fire_idx(t + 2, (k + 2) % 4)
        fire_e(t + 2, (k + 2) % 4)
      if not isinstance(t, int) or t + 1 < nchunk:
        fire_ga((k + 1) % 4)
      pltpu.make_async_copy(h_hbm.at[src_v[r]], rows_v[r], gsem[r]).wait()
      compute(r)
      pltpu.async_copy(rows_v[r], agg_sh.at[dst_v[r]], ssem[r], add=True)

    fire_idx(0, 0)
    fire_e(0, 0)
    fire_idx(1, 1)
    fire_e(1, 1)
    fire_ga(0)
    step(0, 0)
    step(1, 1)

    nquad = (nchunk - 4) // 4

    def quad_body(i, carry):
      t = 2 + 4 * i
      step(t, 2)
      step(t + 1, 3)
      step(t + 2, 0)
      step(t + 3, 1)
      return carry

    lax.fori_loop(0, nquad, quad_body, 0, unroll=False)

    for t in range(2 + 4 * nquad, nchunk):
      step(t, t % 4)

    pltpu.make_async_copy(rows_v[(nchunk - 2) % 4],
                          agg_sh.at[dst_v[(nchunk - 2) % 4]],
                          ssem[(nchunk - 2) % 4]).wait()
    pltpu.make_async_copy(rows_v[(nchunk - 1) % 4],
                          agg_sh.at[dst_v[(nchunk - 1) % 4]],
                          ssem[(nchunk - 1) % 4]).wait()

    plsc.subcore_barrier()

    @pl.when(sid < zero_tiles)
    def _dump():
      pltpu.sync_copy(agg_sh.at[pl.ds(zrow, rows_per_tile)],
                      out_hbm.at[cid].at[pl.ds(zrow, rows_per_tile)])

  return edge_pass


def kernel(x, edge_index, edge_attr, batch, params):
  n, d = x.shape
  e_cnt = edge_index.shape[1]
  zero_blk = jnp.zeros((n // 10, d), jnp.float32)

  edge_pass = _make_edge_pass(n, e_cnt, d, chunk=80)

  def mlp_params(l):
    return (
        params['eps%d' % l].reshape(1, 1),
        params['W1_%d' % l], params['b1_%d' % l].reshape(1, -1),
        params['gm%d' % l].reshape(1, -1), params['bm%d' % l].reshape(1, -1),
        params['W2_%d' % l], params['b2_%d' % l].reshape(1, -1),
        params['g%d' % l].reshape(1, -1), params['b%d' % l].reshape(1, -1),
    )

  src, dst, h, e0 = _k1(
      edge_index, x, params['W_atom'], params['b_atom'].reshape(1, -1),
      edge_attr, params['Wb0'], params['bb0'].reshape(1, -1))
  e1 = _eproj(edge_attr, params['Wb1'], params['bb1'].reshape(1, -1))
  agg2 = edge_pass(h, e0, src, dst, zero_blk)
  h = _node_mlp(h, agg2, *mlp_params(0), final_relu=True)
  agg2 = edge_pass(h, e1, src, dst, zero_blk)
  return _node_mlp(h, agg2, *mlp_params(1), final_relu=False)
